# baseline (speedup 1.0000x reference)
import functools

import jax
import jax.numpy as jnp
import numpy as np
from jax import lax
from jax.experimental import pallas as pl
from jax.experimental.pallas import tpu as pltpu
from jax.experimental.pallas import tpu_sc as plsc

B = 16384
DIM = 64
N_TYPES = 8
N_ITEMS = B * N_TYPES
NUM_CONT = 16
TYPE_SIZE = 100000

NC = 2
NS = 16
NW = NC * NS
ITEMS_PER_W = N_ITEMS // NW
W_PER_TYPE = NW // N_TYPES
B_PER_W = B // W_PER_TYPE
CHUNK = 512
N_CHUNKS = B_PER_W // CHUNK
IDX_MINOR = 128
IDX_ROWS = CHUNK // IDX_MINOR

_CONSTS = np.zeros((8, 128), dtype=np.int32)
_CONSTS[0, :16] = np.arange(16)
_CONSTS[1, :16] = 16


def _sc_gather(ids, table2, consts):
    mesh = plsc.VectorSubcoreMesh(core_axis_name="c", subcore_axis_name="s")

    @functools.partial(
        pl.kernel,
        out_type=jax.ShapeDtypeStruct((N_TYPES, B, 2 * DIM), jnp.float32),
        mesh=mesh,
        scratch_types=[
            pltpu.VMEM((8, 128), jnp.int32),
            pltpu.VMEM((CHUNK, N_TYPES), jnp.int32),
            pltpu.VMEM((IDX_ROWS, IDX_MINOR), jnp.int32),
            pltpu.VMEM((CHUNK, 2 * DIM), jnp.float32),
            pltpu.SemaphoreType.DMA,
        ],
        compiler_params=pltpu.CompilerParams(
            use_tc_tiling_on_sc=False, needs_layout_passes=False
        ),
    )
    def k(ids_hbm, table_hbm, consts_hbm, out_hbm, consts_v, raw_v, idx_v, rows_v, sem):
        wid = lax.axis_index("s") * NC + lax.axis_index("c")
        t = wid // W_PER_TYPE
        bq = wid % W_PER_TYPE
        pltpu.sync_copy(consts_hbm, consts_v)
        iota16 = consts_v[0, pl.ds(0, 16)]
        step16 = consts_v[1, pl.ds(0, 16)]
        tvec = jnp.full((16,), t, dtype=jnp.int32)
        offs = jnp.full((16,), t * TYPE_SIZE, dtype=jnp.int32)
        for c in range(N_CHUNKS):
            b0 = pl.multiple_of(bq * B_PER_W + c * CHUNK, CHUNK)
            pltpu.sync_copy(ids_hbm.at[pl.ds(b0, CHUNK)], raw_v)
            rvec = iota16
            for s in range(CHUNK // 16):
                v = plsc.load_gather(raw_v, [rvec, tvec])
                pair = lax.shift_right_logical(v + offs, 1)
                idx_v[s // N_TYPES, pl.ds((s % N_TYPES) * 16, 16)] = pair
                rvec = rvec + step16
            descs = []
            for i in range(IDX_ROWS):
                descs.append(pltpu.async_copy(
                    table_hbm.at[idx_v.at[i]],
                    rows_v.at[pl.ds(i * IDX_MINOR, IDX_MINOR)],
                    sem,
                ))
            for d in descs:
                d.wait()
            pltpu.sync_copy(rows_v, out_hbm.at[t, pl.ds(b0, CHUNK)])

    return k(ids, table2, consts)


def _tc_assemble(disc, ids, ca, ct):
    bs = 512

    def body(disc_ref, ids_ref, ca_ref, ct_ref, out_ref):
        for t in range(N_TYPES):
            pair = disc_ref[t]
            odd = (ids_ref[:, t] & 1) == 1
            sel = jnp.where(odd[:, None], pair[:, DIM:], pair[:, 0:DIM])
            out_ref[t] = sel.T
        ca_t = ca_ref[...].T
        out_ref[N_TYPES:] = ct_ref[...][:, :, None] * ca_t[:, None, :]

    return pl.pallas_call(
        body,
        grid=(B // bs,),
        in_specs=[
            pl.BlockSpec((N_TYPES, bs, 2 * DIM), lambda i: (0, i, 0)),
            pl.BlockSpec((bs, N_TYPES), lambda i: (i, 0)),
            pl.BlockSpec((bs, NUM_CONT), lambda i: (i, 0)),
            pl.BlockSpec((NUM_CONT, DIM), lambda i: (0, 0)),
        ],
        out_specs=pl.BlockSpec(
            (N_TYPES + NUM_CONT, DIM, bs), lambda i: (0, 0, i)
        ),
        out_shape=jax.ShapeDtypeStruct((N_TYPES + NUM_CONT, DIM, B), jnp.float32),
    )(disc, ids, ca, ct)


def kernel(discrete_actions, continuous_actions, discrete_table, continuous_table):
    consts = jnp.asarray(_CONSTS)
    table2 = discrete_table.reshape(TYPE_SIZE * N_TYPES // 2, 2 * DIM)
    rows = _sc_gather(discrete_actions, table2, consts)
    out_t = _tc_assemble(rows, discrete_actions, continuous_actions, continuous_table)
    return out_t.transpose(2, 0, 1)


# speedup vs baseline: 1.0863x; 1.0863x over previous
import functools

import jax
import jax.numpy as jnp
import numpy as np
from jax import lax
from jax.experimental import pallas as pl
from jax.experimental.pallas import tpu as pltpu
from jax.experimental.pallas import tpu_sc as plsc

B = 16384
DIM = 64
N_TYPES = 8
N_ITEMS = B * N_TYPES
NUM_CONT = 16
TYPE_SIZE = 100000

NC = 2
NS = 16
NW = NC * NS
ITEMS_PER_W = N_ITEMS // NW
W_PER_TYPE = NW // N_TYPES
B_PER_W = B // W_PER_TYPE
CHUNK = 512
N_CHUNKS = B_PER_W // CHUNK
IDX_MINOR = 128
IDX_ROWS = CHUNK // IDX_MINOR

_CONSTS = np.zeros((8, 128), dtype=np.int32)
_CONSTS[0, :16] = np.arange(16)
_CONSTS[1, :16] = 16


def _sc_gather(ids, table, consts):
    mesh = plsc.VectorSubcoreMesh(core_axis_name="c", subcore_axis_name="s")

    @functools.partial(
        pl.kernel,
        out_type=jax.ShapeDtypeStruct((N_TYPES, B, 2 * DIM), jnp.float32),
        mesh=mesh,
        scratch_types=[
            pltpu.VMEM((8, 128), jnp.int32),
            pltpu.VMEM((CHUNK, N_TYPES), jnp.int32),
            pltpu.VMEM((IDX_ROWS, IDX_MINOR), jnp.int32),
            pltpu.VMEM((CHUNK, DIM), jnp.float32),
            pltpu.SemaphoreType.DMA,
        ],
        compiler_params=pltpu.CompilerParams(
            use_tc_tiling_on_sc=False, needs_layout_passes=False
        ),
    )
    def k(ids_hbm, table_hbm, consts_hbm, out_hbm, consts_v, raw_v, idx_v, rows_v, sem):
        wid = lax.axis_index("s") * NC + lax.axis_index("c")
        t = wid // W_PER_TYPE
        bq = wid % W_PER_TYPE
        pltpu.sync_copy(consts_hbm, consts_v)
        iota16 = consts_v[0, pl.ds(0, 16)]
        step16 = consts_v[1, pl.ds(0, 16)]
        tvec = jnp.full((16,), t, dtype=jnp.int32)
        offs = jnp.full((16,), t * TYPE_SIZE, dtype=jnp.int32)
        for c in range(N_CHUNKS):
            b0 = pl.multiple_of(bq * B_PER_W + c * CHUNK, CHUNK)
            pltpu.sync_copy(ids_hbm.at[pl.ds(b0, CHUNK)], raw_v)
            rvec = iota16
            for s in range(CHUNK // 16):
                v = plsc.load_gather(raw_v, [rvec, tvec])
                idx_v[s // N_TYPES, pl.ds((s % N_TYPES) * 16, 16)] = v + offs
                rvec = rvec + step16
            descs = []
            for i in range(IDX_ROWS):
                descs.append(pltpu.async_copy(
                    table_hbm.at[idx_v.at[i]],
                    rows_v.at[pl.ds(i * IDX_MINOR, IDX_MINOR)],
                    sem,
                ))
            for d in descs:
                d.wait()
            pltpu.sync_copy(
                rows_v, out_hbm.at[t, pl.ds(b0, CHUNK), pl.ds(0, DIM)]
            )

    return k(ids, table, consts)


def _tc_cont(ca_t, ct):
    bs = 1024

    def body(ca_ref, ct_ref, out_ref):
        out_ref[...] = ct_ref[...][:, :, None] * ca_ref[...][:, None, :]

    return pl.pallas_call(
        body,
        grid=(2, B // bs),
        in_specs=[
            pl.BlockSpec((N_TYPES, bs), lambda j, i: (j, i)),
            pl.BlockSpec((N_TYPES, DIM), lambda j, i: (j, 0)),
        ],
        out_specs=pl.BlockSpec(
            (N_TYPES, DIM, bs), lambda j, i: (j + 1, 0, i)
        ),
        out_shape=jax.ShapeDtypeStruct((N_TYPES + NUM_CONT, DIM, B), jnp.float32),
    )(ca_t, ct)


def _tc_disc(disc, prev):
    bs = 512

    def body(disc_ref, prev_ref, out_ref):
        del prev_ref
        for t in range(N_TYPES):
            out_ref[t] = disc_ref[t, :, 0:DIM].T

    return pl.pallas_call(
        body,
        grid=(B // bs,),
        in_specs=[
            pl.BlockSpec((N_TYPES, bs, 2 * DIM), lambda i: (0, i, 0)),
            pl.BlockSpec((N_TYPES, DIM, bs), lambda i: (0, 0, i)),
        ],
        out_specs=pl.BlockSpec((N_TYPES, DIM, bs), lambda i: (0, 0, i)),
        out_shape=jax.ShapeDtypeStruct((N_TYPES + NUM_CONT, DIM, B), jnp.float32),
        input_output_aliases={1: 0},
    )(disc, prev)


def kernel(discrete_actions, continuous_actions, discrete_table, continuous_table):
    consts = jnp.asarray(_CONSTS)
    rows = _sc_gather(discrete_actions, discrete_table, consts)
    out_cont = _tc_cont(continuous_actions.T, continuous_table)
    out_t = _tc_disc(rows, out_cont)
    return out_t.transpose(2, 0, 1)


# speedup vs baseline: 1.4981x; 1.3790x over previous
import functools

import jax
import jax.numpy as jnp
import numpy as np
from jax import lax
from jax.experimental import pallas as pl
from jax.experimental.pallas import tpu as pltpu
from jax.experimental.pallas import tpu_sc as plsc

B = 16384
DIM = 64
N_TYPES = 8
N_ITEMS = B * N_TYPES
NUM_CONT = 16
TYPE_SIZE = 100000

NC = 2
NS = 16
NW = NC * NS
ITEMS_PER_W = N_ITEMS // NW
W_PER_TYPE = NW // N_TYPES
B_PER_W = B // W_PER_TYPE
CHUNK = 512
N_CHUNKS = B_PER_W // CHUNK
IDX_MINOR = 128
IDX_ROWS = CHUNK // IDX_MINOR

_CONSTS = np.zeros((8, 128), dtype=np.int32)
_CONSTS[0, :16] = np.arange(16)
_CONSTS[1, :16] = 16


HALF = TYPE_SIZE * N_TYPES // 2


def _tc_flatten(table_t):
    bsr = 2048

    def body(a_ref, b_ref, out_ref):
        out_ref[:, 0:DIM] = a_ref[...].T
        out_ref[:, DIM:] = b_ref[...].T

    return pl.pallas_call(
        body,
        grid=(HALF // bsr,),
        in_specs=[
            pl.BlockSpec((DIM, bsr), lambda i: (0, i)),
            pl.BlockSpec((DIM, bsr), lambda i: (0, i + HALF // bsr)),
        ],
        out_specs=pl.BlockSpec((bsr, 2 * DIM), lambda i: (i, 0)),
        out_shape=jax.ShapeDtypeStruct((HALF, 2 * DIM), jnp.float32),
    )(table_t, table_t)


def _sc_gather(ids, table2, consts):
    mesh = plsc.VectorSubcoreMesh(core_axis_name="c", subcore_axis_name="s")

    @functools.partial(
        pl.kernel,
        out_type=jax.ShapeDtypeStruct((N_TYPES, B, 2 * DIM), jnp.float32),
        mesh=mesh,
        scratch_types=[
            pltpu.VMEM((8, 128), jnp.int32),
            pltpu.VMEM((CHUNK, N_TYPES), jnp.int32),
            pltpu.VMEM((IDX_ROWS, IDX_MINOR), jnp.int32),
            pltpu.VMEM((CHUNK, 2 * DIM), jnp.float32),
            pltpu.SemaphoreType.DMA,
        ],
        compiler_params=pltpu.CompilerParams(
            use_tc_tiling_on_sc=False, needs_layout_passes=False
        ),
    )
    def k(ids_hbm, table_hbm, consts_hbm, out_hbm, consts_v, raw_v, idx_v, rows_v, sem):
        wid = lax.axis_index("s") * NC + lax.axis_index("c")
        t = wid // W_PER_TYPE
        bq = wid % W_PER_TYPE
        pltpu.sync_copy(consts_hbm, consts_v)
        iota16 = consts_v[0, pl.ds(0, 16)]
        step16 = consts_v[1, pl.ds(0, 16)]
        tvec = jnp.full((16,), t, dtype=jnp.int32)
        offs = jnp.full((16,), (t % 4) * TYPE_SIZE, dtype=jnp.int32)
        for c in range(N_CHUNKS):
            b0 = pl.multiple_of(bq * B_PER_W + c * CHUNK, CHUNK)
            pltpu.sync_copy(ids_hbm.at[pl.ds(b0, CHUNK)], raw_v)
            rvec = iota16
            for s in range(CHUNK // 16):
                v = plsc.load_gather(raw_v, [rvec, tvec])
                idx_v[s // N_TYPES, pl.ds((s % N_TYPES) * 16, 16)] = v + offs
                rvec = rvec + step16
            descs = []
            for i in range(IDX_ROWS):
                descs.append(pltpu.async_copy(
                    table_hbm.at[idx_v.at[i]],
                    rows_v.at[pl.ds(i * IDX_MINOR, IDX_MINOR)],
                    sem,
                ))
            for d in descs:
                d.wait()
            pltpu.sync_copy(rows_v, out_hbm.at[t, pl.ds(b0, CHUNK)])

    return k(ids, table2, consts)


def _tc_cont(ca_t, ct):
    bs = 1024

    def body(ca_ref, ct_ref, out_ref):
        out_ref[...] = ct_ref[...][:, :, None] * ca_ref[...][:, None, :]

    return pl.pallas_call(
        body,
        grid=(2, B // bs),
        in_specs=[
            pl.BlockSpec((N_TYPES, bs), lambda j, i: (j, i)),
            pl.BlockSpec((N_TYPES, DIM), lambda j, i: (j, 0)),
        ],
        out_specs=pl.BlockSpec(
            (N_TYPES, DIM, bs), lambda j, i: (j + 1, 0, i)
        ),
        out_shape=jax.ShapeDtypeStruct((N_TYPES + NUM_CONT, DIM, B), jnp.float32),
    )(ca_t, ct)


def _tc_disc(disc, prev):
    bs = 512

    def body(disc_ref, prev_ref, out_ref):
        del prev_ref
        for t in range(N_TYPES):
            lo = (t // 4) * DIM
            out_ref[t] = disc_ref[t, :, lo:lo + DIM].T

    return pl.pallas_call(
        body,
        grid=(B // bs,),
        in_specs=[
            pl.BlockSpec((N_TYPES, bs, 2 * DIM), lambda i: (0, i, 0)),
            pl.BlockSpec((N_TYPES, DIM, bs), lambda i: (0, 0, i)),
        ],
        out_specs=pl.BlockSpec((N_TYPES, DIM, bs), lambda i: (0, 0, i)),
        out_shape=jax.ShapeDtypeStruct((N_TYPES + NUM_CONT, DIM, B), jnp.float32),
        input_output_aliases={1: 0},
    )(disc, prev)


def kernel(discrete_actions, continuous_actions, discrete_table, continuous_table):
    consts = jnp.asarray(_CONSTS)
    table2 = _tc_flatten(discrete_table.T)
    rows = _sc_gather(discrete_actions, table2, consts)
    out_cont = _tc_cont(continuous_actions.T, continuous_table)
    out_t = _tc_disc(rows, out_cont)
    return out_t.transpose(2, 0, 1)


# speedup vs baseline: 1.6601x; 1.1081x over previous
import functools

import jax
import jax.numpy as jnp
import numpy as np
from jax import lax
from jax.experimental import pallas as pl
from jax.experimental.pallas import tpu as pltpu
from jax.experimental.pallas import tpu_sc as plsc

B = 16384
DIM = 64
N_TYPES = 8
N_ITEMS = B * N_TYPES
NUM_CONT = 16
TYPE_SIZE = 100000

NC = 2
NS = 16
NW = NC * NS
ITEMS_PER_W = N_ITEMS // NW
W_PER_TYPE = NW // N_TYPES
B_PER_W = B // W_PER_TYPE
CHUNK = 512
N_CHUNKS = B_PER_W // CHUNK
IDX_MINOR = 128
IDX_ROWS = CHUNK // IDX_MINOR

_CONSTS = np.zeros((8, 128), dtype=np.int32)
_CONSTS[0, :16] = np.arange(16)
_CONSTS[1, :16] = 16


HALF = TYPE_SIZE * N_TYPES // 2


def _tc_flatten(table_t):
    bsr = 3200

    def body(a_ref, b_ref, out_ref):
        out_ref[:, 0:DIM] = a_ref[...].T
        out_ref[:, DIM:] = b_ref[...].T

    return pl.pallas_call(
        body,
        grid=(HALF // bsr,),
        in_specs=[
            pl.BlockSpec((DIM, bsr), lambda i: (0, i)),
            pl.BlockSpec((DIM, bsr), lambda i: (0, i + HALF // bsr)),
        ],
        out_specs=pl.BlockSpec((bsr, 2 * DIM), lambda i: (i, 0)),
        out_shape=jax.ShapeDtypeStruct((HALF, 2 * DIM), jnp.float32),
    )(table_t, table_t)


def _sc_gather(ids, table2, consts):
    mesh = plsc.VectorSubcoreMesh(core_axis_name="c", subcore_axis_name="s")

    @functools.partial(
        pl.kernel,
        out_type=jax.ShapeDtypeStruct((N_TYPES, B, 2 * DIM), jnp.float32),
        mesh=mesh,
        scratch_types=[
            pltpu.VMEM((8, 128), jnp.int32),
            pltpu.VMEM((CHUNK, N_TYPES), jnp.int32),
            pltpu.VMEM((IDX_ROWS, IDX_MINOR), jnp.int32),
            pltpu.VMEM((CHUNK, 2 * DIM), jnp.float32),
            pltpu.SemaphoreType.DMA,
        ],
        compiler_params=pltpu.CompilerParams(
            use_tc_tiling_on_sc=False, needs_layout_passes=False
        ),
    )
    def k(ids_hbm, table_hbm, consts_hbm, out_hbm, consts_v, raw_v, idx_v, rows_v, sem):
        wid = lax.axis_index("s") * NC + lax.axis_index("c")
        t = wid // W_PER_TYPE
        bq = wid % W_PER_TYPE
        pltpu.sync_copy(consts_hbm, consts_v)
        iota16 = consts_v[0, pl.ds(0, 16)]
        step16 = consts_v[1, pl.ds(0, 16)]
        tvec = jnp.full((16,), t, dtype=jnp.int32)
        offs = jnp.full((16,), (t % 4) * TYPE_SIZE, dtype=jnp.int32)
        for c in range(N_CHUNKS):
            b0 = pl.multiple_of(bq * B_PER_W + c * CHUNK, CHUNK)
            pltpu.sync_copy(ids_hbm.at[pl.ds(b0, CHUNK)], raw_v)
            rvec = iota16
            for s in range(CHUNK // 16):
                v = plsc.load_gather(raw_v, [rvec, tvec])
                idx_v[s // N_TYPES, pl.ds((s % N_TYPES) * 16, 16)] = v + offs
                rvec = rvec + step16
            descs = []
            for i in range(IDX_ROWS):
                descs.append(pltpu.async_copy(
                    table_hbm.at[idx_v.at[i]],
                    rows_v.at[pl.ds(i * IDX_MINOR, IDX_MINOR)],
                    sem,
                ))
            for d in descs:
                d.wait()
            pltpu.sync_copy(rows_v, out_hbm.at[t, pl.ds(b0, CHUNK)])

    return k(ids, table2, consts)


def _tc_cont(ca_t, ct):
    bs = 1024

    def body(ca_ref, ct_ref, out_ref):
        out_ref[...] = ct_ref[...][:, :, None] * ca_ref[...][:, None, :]

    return pl.pallas_call(
        body,
        grid=(2, B // bs),
        in_specs=[
            pl.BlockSpec((N_TYPES, bs), lambda j, i: (j, i)),
            pl.BlockSpec((N_TYPES, DIM), lambda j, i: (j, 0)),
        ],
        out_specs=pl.BlockSpec(
            (N_TYPES, DIM, bs), lambda j, i: (j + 1, 0, i)
        ),
        out_shape=jax.ShapeDtypeStruct((N_TYPES + NUM_CONT, DIM, B), jnp.float32),
    )(ca_t, ct)


def _tc_disc(disc, prev):
    bs = 512

    def body(disc_ref, prev_ref, out_ref):
        del prev_ref
        for t in range(N_TYPES):
            lo = (t // 4) * DIM
            out_ref[t] = disc_ref[t, :, lo:lo + DIM].T

    return pl.pallas_call(
        body,
        grid=(B // bs,),
        in_specs=[
            pl.BlockSpec((N_TYPES, bs, 2 * DIM), lambda i: (0, i, 0)),
            pl.BlockSpec((N_TYPES, DIM, bs), lambda i: (0, 0, i)),
        ],
        out_specs=pl.BlockSpec((N_TYPES, DIM, bs), lambda i: (0, 0, i)),
        out_shape=jax.ShapeDtypeStruct((N_TYPES + NUM_CONT, DIM, B), jnp.float32),
        input_output_aliases={1: 0},
    )(disc, prev)


def kernel(discrete_actions, continuous_actions, discrete_table, continuous_table):
    consts = jnp.asarray(_CONSTS)
    table2 = _tc_flatten(discrete_table.T)
    rows = _sc_gather(discrete_actions, table2, consts)
    out_cont = _tc_cont(continuous_actions.T, continuous_table)
    out_t = _tc_disc(rows, out_cont)
    return out_t.transpose(2, 0, 1)


# speedup vs baseline: 1.9207x; 1.1570x over previous
import functools

import jax
import jax.numpy as jnp
import numpy as np
from jax import lax
from jax.experimental import pallas as pl
from jax.experimental.pallas import tpu as pltpu
from jax.experimental.pallas import tpu_sc as plsc

B = 16384
DIM = 64
N_TYPES = 8
N_ITEMS = B * N_TYPES
NUM_CONT = 16
TYPE_SIZE = 100000

NC = 2
NS = 16
NW = NC * NS
ITEMS_PER_W = N_ITEMS // NW
W_PER_TYPE = NW // N_TYPES
B_PER_W = B // W_PER_TYPE
CHUNK = 512
N_CHUNKS = B_PER_W // CHUNK
IDX_MINOR = 128
IDX_ROWS = CHUNK // IDX_MINOR

_CONSTS = np.zeros((8, 128), dtype=np.int32)
_CONSTS[0, :16] = np.arange(16)
_CONSTS[1, :16] = 16


HALF = TYPE_SIZE * N_TYPES // 2


def _tc_flatten(table_t):
    bsr = 16000

    def body(a_ref, b_ref, out_ref):
        out_ref[:, 0:DIM] = a_ref[...].T
        out_ref[:, DIM:] = b_ref[...].T

    return pl.pallas_call(
        body,
        grid=(HALF // bsr,),
        in_specs=[
            pl.BlockSpec((DIM, bsr), lambda i: (0, i)),
            pl.BlockSpec((DIM, bsr), lambda i: (0, i + HALF // bsr)),
        ],
        out_specs=pl.BlockSpec((bsr, 2 * DIM), lambda i: (i, 0)),
        out_shape=jax.ShapeDtypeStruct((HALF, 2 * DIM), jnp.float32),
    )(table_t, table_t)


def _sc_gather(ids, table2, consts):
    mesh = plsc.VectorSubcoreMesh(core_axis_name="c", subcore_axis_name="s")

    @functools.partial(
        pl.kernel,
        out_type=jax.ShapeDtypeStruct((N_TYPES, B, 2 * DIM), jnp.float32),
        mesh=mesh,
        scratch_types=[
            pltpu.VMEM((8, 128), jnp.int32),
            pltpu.VMEM((CHUNK, N_TYPES), jnp.int32),
            pltpu.VMEM((IDX_ROWS, IDX_MINOR), jnp.int32),
            pltpu.VMEM((CHUNK, 2 * DIM), jnp.float32),
            pltpu.SemaphoreType.DMA,
        ],
        compiler_params=pltpu.CompilerParams(
            use_tc_tiling_on_sc=False, needs_layout_passes=False
        ),
    )
    def k(ids_hbm, table_hbm, consts_hbm, out_hbm, consts_v, raw_v, idx_v, rows_v, sem):
        wid = lax.axis_index("s") * NC + lax.axis_index("c")
        t = wid // W_PER_TYPE
        bq = wid % W_PER_TYPE
        pltpu.sync_copy(consts_hbm, consts_v)
        iota16 = consts_v[0, pl.ds(0, 16)]
        step16 = consts_v[1, pl.ds(0, 16)]
        tvec = jnp.full((16,), t, dtype=jnp.int32)
        offs = jnp.full((16,), (t % 4) * TYPE_SIZE, dtype=jnp.int32)
        for c in range(N_CHUNKS):
            b0 = pl.multiple_of(bq * B_PER_W + c * CHUNK, CHUNK)
            pltpu.sync_copy(ids_hbm.at[pl.ds(b0, CHUNK)], raw_v)
            rvec = iota16
            for s in range(CHUNK // 16):
                v = plsc.load_gather(raw_v, [rvec, tvec])
                idx_v[s // N_TYPES, pl.ds((s % N_TYPES) * 16, 16)] = v + offs
                rvec = rvec + step16
            descs = []
            for i in range(IDX_ROWS):
                descs.append(pltpu.async_copy(
                    table_hbm.at[idx_v.at[i]],
                    rows_v.at[pl.ds(i * IDX_MINOR, IDX_MINOR)],
                    sem,
                ))
            for d in descs:
                d.wait()
            pltpu.sync_copy(rows_v, out_hbm.at[t, pl.ds(b0, CHUNK)])

    return k(ids, table2, consts)


def _tc_cont(ca_t, ct):
    bs = 1024

    def body(ca_ref, ct_ref, out_ref):
        out_ref[...] = ct_ref[...][:, :, None] * ca_ref[...][:, None, :]

    return pl.pallas_call(
        body,
        grid=(2, B // bs),
        in_specs=[
            pl.BlockSpec((N_TYPES, bs), lambda j, i: (j, i)),
            pl.BlockSpec((N_TYPES, DIM), lambda j, i: (j, 0)),
        ],
        out_specs=pl.BlockSpec(
            (N_TYPES, DIM, bs), lambda j, i: (j + 1, 0, i)
        ),
        out_shape=jax.ShapeDtypeStruct((N_TYPES + NUM_CONT, DIM, B), jnp.float32),
    )(ca_t, ct)


def _tc_disc(disc, prev):
    bs = 512

    def body(disc_ref, prev_ref, out_ref):
        del prev_ref
        for t in range(N_TYPES):
            lo = (t // 4) * DIM
            out_ref[t] = disc_ref[t, :, lo:lo + DIM].T

    return pl.pallas_call(
        body,
        grid=(B // bs,),
        in_specs=[
            pl.BlockSpec((N_TYPES, bs, 2 * DIM), lambda i: (0, i, 0)),
            pl.BlockSpec((N_TYPES, DIM, bs), lambda i: (0, 0, i)),
        ],
        out_specs=pl.BlockSpec((N_TYPES, DIM, bs), lambda i: (0, 0, i)),
        out_shape=jax.ShapeDtypeStruct((N_TYPES + NUM_CONT, DIM, B), jnp.float32),
        input_output_aliases={1: 0},
    )(disc, prev)


def kernel(discrete_actions, continuous_actions, discrete_table, continuous_table):
    consts = jnp.asarray(_CONSTS)
    table2 = _tc_flatten(discrete_table.T)
    rows = _sc_gather(discrete_actions, table2, consts)
    out_cont = _tc_cont(continuous_actions.T, continuous_table)
    out_t = _tc_disc(rows, out_cont)
    return out_t.transpose(2, 0, 1)


# speedup vs baseline: 1.9435x; 1.0119x over previous
import functools

import jax
import jax.numpy as jnp
import numpy as np
from jax import lax
from jax.experimental import pallas as pl
from jax.experimental.pallas import tpu as pltpu
from jax.experimental.pallas import tpu_sc as plsc

B = 16384
DIM = 64
N_TYPES = 8
N_ITEMS = B * N_TYPES
NUM_CONT = 16
TYPE_SIZE = 100000

NC = 2
NS = 16
NW = NC * NS
ITEMS_PER_W = N_ITEMS // NW
W_PER_TYPE = NW // N_TYPES
B_PER_W = B // W_PER_TYPE
CHUNK = 512
N_CHUNKS = B_PER_W // CHUNK
IDX_MINOR = 128
IDX_ROWS = CHUNK // IDX_MINOR

_CONSTS = np.zeros((8, 128), dtype=np.int32)
_CONSTS[0, :16] = np.arange(16)
_CONSTS[1, :16] = 16


HALF = TYPE_SIZE * N_TYPES // 2


def _tc_flatten(table_t):
    bsr = 16000

    def body(a_ref, b_ref, out_ref):
        out_ref[:, 0:DIM] = a_ref[...].T
        out_ref[:, DIM:] = b_ref[...].T

    return pl.pallas_call(
        body,
        grid=(HALF // bsr,),
        in_specs=[
            pl.BlockSpec((DIM, bsr), lambda i: (0, i)),
            pl.BlockSpec((DIM, bsr), lambda i: (0, i + HALF // bsr)),
        ],
        out_specs=pl.BlockSpec((bsr, 2 * DIM), lambda i: (i, 0)),
        out_shape=jax.ShapeDtypeStruct((HALF, 2 * DIM), jnp.float32),
    )(table_t, table_t)


def _sc_gather(ids, table2, consts):
    mesh = plsc.VectorSubcoreMesh(core_axis_name="c", subcore_axis_name="s")

    @functools.partial(
        pl.kernel,
        out_type=jax.ShapeDtypeStruct((N_TYPES, B, 2 * DIM), jnp.float32),
        mesh=mesh,
        scratch_types=[
            pltpu.VMEM((8, 128), jnp.int32),
            pltpu.VMEM((CHUNK, N_TYPES), jnp.int32),
            pltpu.VMEM((IDX_ROWS, IDX_MINOR), jnp.int32),
            pltpu.VMEM((CHUNK, 2 * DIM), jnp.float32),
            pltpu.SemaphoreType.DMA,
        ],
        compiler_params=pltpu.CompilerParams(
            use_tc_tiling_on_sc=False, needs_layout_passes=False
        ),
    )
    def k(ids_hbm, table_hbm, consts_hbm, out_hbm, consts_v, raw_v, idx_v, rows_v, sem):
        wid = lax.axis_index("s") * NC + lax.axis_index("c")
        t = wid // W_PER_TYPE
        bq = wid % W_PER_TYPE
        pltpu.sync_copy(consts_hbm, consts_v)
        iota16 = consts_v[0, pl.ds(0, 16)]
        step16 = consts_v[1, pl.ds(0, 16)]
        tvec = jnp.full((16,), t, dtype=jnp.int32)
        offs = jnp.full((16,), (t % 4) * TYPE_SIZE, dtype=jnp.int32)
        for c in range(N_CHUNKS):
            b0 = pl.multiple_of(bq * B_PER_W + c * CHUNK, CHUNK)
            pltpu.sync_copy(ids_hbm.at[pl.ds(b0, CHUNK)], raw_v)
            rvec = iota16
            for s in range(CHUNK // 16):
                v = plsc.load_gather(raw_v, [rvec, tvec])
                idx_v[s // N_TYPES, pl.ds((s % N_TYPES) * 16, 16)] = v + offs
                rvec = rvec + step16
            descs = []
            for i in range(IDX_ROWS):
                descs.append(pltpu.async_copy(
                    table_hbm.at[idx_v.at[i]],
                    rows_v.at[pl.ds(i * IDX_MINOR, IDX_MINOR)],
                    sem,
                ))
            for d in descs:
                d.wait()
            pltpu.sync_copy(rows_v, out_hbm.at[t, pl.ds(b0, CHUNK)])

    return k(ids, table2, consts)


def _tc_cont(ca_t, ct):
    bs = 1024

    def body(ca_ref, ct_ref, out_ref):
        out_ref[...] = ct_ref[...][:, :, None] * ca_ref[...][:, None, :]

    return pl.pallas_call(
        body,
        grid=(2, B // bs),
        in_specs=[
            pl.BlockSpec((N_TYPES, bs), lambda j, i: (j, i)),
            pl.BlockSpec((N_TYPES, DIM), lambda j, i: (j, 0)),
        ],
        out_specs=pl.BlockSpec(
            (N_TYPES, DIM, bs), lambda j, i: (j + 1, 0, i)
        ),
        out_shape=jax.ShapeDtypeStruct((N_TYPES + NUM_CONT, DIM, B), jnp.float32),
    )(ca_t, ct)


def _tc_disc(disc, prev):
    bs = 512

    def body(disc_ref, prev_ref, out_ref):
        del prev_ref
        for t in range(N_TYPES):
            lo = (t // 4) * DIM
            out_ref[t] = disc_ref[t, :, lo:lo + DIM].T

    return pl.pallas_call(
        body,
        grid=(B // bs,),
        in_specs=[
            pl.BlockSpec((N_TYPES, bs, 2 * DIM), lambda i: (0, i, 0)),
            pl.BlockSpec(memory_space=pltpu.MemorySpace.HBM),
        ],
        out_specs=pl.BlockSpec((N_TYPES, DIM, bs), lambda i: (0, 0, i)),
        out_shape=jax.ShapeDtypeStruct((N_TYPES + NUM_CONT, DIM, B), jnp.float32),
        input_output_aliases={1: 0},
    )(disc, prev)


def kernel(discrete_actions, continuous_actions, discrete_table, continuous_table):
    consts = jnp.asarray(_CONSTS)
    table2 = _tc_flatten(discrete_table.T)
    rows = _sc_gather(discrete_actions, table2, consts)
    out_cont = _tc_cont(continuous_actions.T, continuous_table)
    out_t = _tc_disc(rows, out_cont)
    return out_t.transpose(2, 0, 1)


# speedup vs baseline: 1.9993x; 1.0287x over previous
import functools

import jax
import jax.numpy as jnp
import numpy as np
from jax import lax
from jax.experimental import pallas as pl
from jax.experimental.pallas import tpu as pltpu
from jax.experimental.pallas import tpu_sc as plsc

B = 16384
DIM = 64
N_TYPES = 8
N_ITEMS = B * N_TYPES
NUM_CONT = 16
TYPE_SIZE = 100000

NC = 2
NS = 16
NW = NC * NS
ITEMS_PER_W = N_ITEMS // NW
W_PER_TYPE = NW // N_TYPES
B_PER_W = B // W_PER_TYPE
CHUNK = 512
N_CHUNKS = B_PER_W // CHUNK
IDX_MINOR = 128
IDX_ROWS = CHUNK // IDX_MINOR

_CONSTS = np.zeros((8, 128), dtype=np.int32)
_CONSTS[0, :16] = np.arange(16)
_CONSTS[1, :16] = 16


HALF = TYPE_SIZE * N_TYPES // 2


def _tc_flatten(table_t):
    bsr = 16000

    def body(a_ref, b_ref, out_ref):
        out_ref[:, 0:DIM] = a_ref[...].T
        out_ref[:, DIM:] = b_ref[...].T

    return pl.pallas_call(
        body,
        grid=(HALF // bsr,),
        in_specs=[
            pl.BlockSpec((DIM, bsr), lambda i: (0, i)),
            pl.BlockSpec((DIM, bsr), lambda i: (0, i + HALF // bsr)),
        ],
        out_specs=pl.BlockSpec((bsr, 2 * DIM), lambda i: (i, 0)),
        out_shape=jax.ShapeDtypeStruct((HALF, 2 * DIM), jnp.float32),
    )(table_t, table_t)


def _sc_gather(ids, table2, consts):
    mesh = plsc.VectorSubcoreMesh(core_axis_name="c", subcore_axis_name="s")

    @functools.partial(
        pl.kernel,
        out_type=jax.ShapeDtypeStruct((N_TYPES, B, 2 * DIM), jnp.float32),
        mesh=mesh,
        scratch_types=[
            pltpu.VMEM((8, 128), jnp.int32),
            pltpu.VMEM((CHUNK, N_TYPES), jnp.int32),
            pltpu.VMEM((IDX_ROWS, IDX_MINOR), jnp.int32),
            pltpu.VMEM((CHUNK, 2 * DIM), jnp.float32),
            pltpu.SemaphoreType.DMA,
        ],
        compiler_params=pltpu.CompilerParams(
            use_tc_tiling_on_sc=False, needs_layout_passes=False
        ),
    )
    def k(ids_hbm, table_hbm, consts_hbm, out_hbm, consts_v, raw_v, idx_v, rows_v, sem):
        wid = lax.axis_index("s") * NC + lax.axis_index("c")
        t = wid // W_PER_TYPE
        bq = wid % W_PER_TYPE
        pltpu.sync_copy(consts_hbm, consts_v)
        iota16 = consts_v[0, pl.ds(0, 16)]
        step16 = consts_v[1, pl.ds(0, 16)]
        tvec = jnp.full((16,), t, dtype=jnp.int32)
        offs = jnp.full((16,), (t % 4) * TYPE_SIZE, dtype=jnp.int32)
        for c in range(N_CHUNKS):
            b0 = pl.multiple_of(bq * B_PER_W + c * CHUNK, CHUNK)
            pltpu.sync_copy(ids_hbm.at[pl.ds(b0, CHUNK)], raw_v)
            rvec = iota16
            for s in range(CHUNK // 16):
                v = plsc.load_gather(raw_v, [rvec, tvec])
                idx_v[s // N_TYPES, pl.ds((s % N_TYPES) * 16, 16)] = v + offs
                rvec = rvec + step16
            descs = []
            for i in range(IDX_ROWS):
                descs.append(pltpu.async_copy(
                    table_hbm.at[idx_v.at[i]],
                    rows_v.at[pl.ds(i * IDX_MINOR, IDX_MINOR)],
                    sem,
                ))
            for d in descs:
                d.wait()
            pltpu.sync_copy(rows_v, out_hbm.at[t, pl.ds(b0, CHUNK)])

    return k(ids, table2, consts)


def _tc_cont(ca_t, ct):
    bs = 1024

    def body(ca_ref, ct_ref, out_ref):
        out_ref[...] = ct_ref[...][:, :, None] * ca_ref[...][:, None, :]

    return pl.pallas_call(
        body,
        grid=(2, B // bs),
        in_specs=[
            pl.BlockSpec((N_TYPES, bs), lambda j, i: (j, i)),
            pl.BlockSpec((N_TYPES, DIM), lambda j, i: (j, 0)),
        ],
        out_specs=pl.BlockSpec(
            (N_TYPES, DIM, bs), lambda j, i: (j + 1, 0, i)
        ),
        out_shape=jax.ShapeDtypeStruct((N_TYPES + NUM_CONT, DIM, B), jnp.float32),
    )(ca_t, ct)


def _tc_disc(disc, prev):
    bs = 1024

    def body(disc_ref, prev_ref, out_ref):
        del prev_ref
        for t in range(N_TYPES):
            lo = (t // 4) * DIM
            out_ref[t] = disc_ref[t, :, lo:lo + DIM].T

    return pl.pallas_call(
        body,
        grid=(B // bs,),
        in_specs=[
            pl.BlockSpec((N_TYPES, bs, 2 * DIM), lambda i: (0, i, 0)),
            pl.BlockSpec(memory_space=pltpu.MemorySpace.HBM),
        ],
        out_specs=pl.BlockSpec((N_TYPES, DIM, bs), lambda i: (0, 0, i)),
        out_shape=jax.ShapeDtypeStruct((N_TYPES + NUM_CONT, DIM, B), jnp.float32),
        input_output_aliases={1: 0},
    )(disc, prev)


def kernel(discrete_actions, continuous_actions, discrete_table, continuous_table):
    consts = jnp.asarray(_CONSTS)
    table2 = _tc_flatten(discrete_table.T)
    rows = _sc_gather(discrete_actions, table2, consts)
    out_cont = _tc_cont(continuous_actions.T, continuous_table)
    out_t = _tc_disc(rows, out_cont)
    return out_t.transpose(2, 0, 1)
